# baseline (device time: 11975 ns/iter reference)
import jax
import jax.numpy as jnp
from jax import lax
from jax.experimental import pallas as pl
from jax.experimental.pallas import tpu as pltpu

N_DEV = 8
N_ROUNDS = 3


def kernel(x):
    m, n = x.shape

    def body(x_ref, out_ref, send_ref, recv_ref, send_sems, recv_sems):
        my = lax.axis_index("i")
        partners = [jnp.bitwise_xor(my, 1 << k) for k in range(N_ROUNDS)]

        barrier_sem = pltpu.get_barrier_semaphore()
        for p in partners:
            pl.semaphore_signal(
                barrier_sem, inc=1,
                device_id=(p,), device_id_type=pl.DeviceIdType.MESH,
            )
        pl.semaphore_wait(barrier_sem, N_ROUNDS)

        def rdma(k):
            return pltpu.make_async_remote_copy(
                src_ref=send_ref.at[k],
                dst_ref=recv_ref.at[k],
                send_sem=send_sems.at[k],
                recv_sem=recv_sems.at[k],
                device_id=(partners[k],),
                device_id_type=pl.DeviceIdType.MESH,
            )

        t = x_ref[:, :].astype(jnp.float32)
        h = m
        while h > 1:
            h //= 2
            t = t[:h, :] * t[h:, :]
        send_ref[0, :, :] = t

        r0 = rdma(0)
        r0.start()

        lc = x_ref[:, :].astype(jnp.float32)
        s = 1
        while s < 32:
            shifted = jnp.concatenate(
                [jnp.ones((s, n), jnp.float32), lc[:-s, :]], axis=0
            )
            lc = lc * shifted
            s *= 2

        r0.wait()
        send_ref[1, :, :] = send_ref[0, :, :] * recv_ref[0, :, :]
        e0 = jnp.where(partners[0] < my, recv_ref[0, :, :], 1.0)
        r1 = rdma(1)
        r1.start()

        while s < m:
            shifted = jnp.concatenate(
                [jnp.ones((s, n), jnp.float32), lc[:-s, :]], axis=0
            )
            lc = lc * shifted
            s *= 2
        lc = lc * e0

        r1.wait()
        send_ref[2, :, :] = send_ref[1, :, :] * recv_ref[1, :, :]
        e1 = jnp.where(partners[1] < my, recv_ref[1, :, :], 1.0)
        r2 = rdma(2)
        r2.start()

        lc = lc * e1

        r2.wait()
        e2 = jnp.where(partners[2] < my, recv_ref[2, :, :], 1.0)
        out_ref[:, :] = lc * e2

    return pl.pallas_call(
        body,
        out_shape=jax.ShapeDtypeStruct((m, n), jnp.float32),
        in_specs=[pl.BlockSpec(memory_space=pltpu.VMEM)],
        out_specs=pl.BlockSpec(memory_space=pltpu.VMEM),
        scratch_shapes=[
            pltpu.VMEM((N_ROUNDS, 1, n), jnp.float32),
            pltpu.VMEM((N_ROUNDS, 1, n), jnp.float32),
            pltpu.SemaphoreType.DMA((N_ROUNDS,)),
            pltpu.SemaphoreType.DMA((N_ROUNDS,)),
        ],
        compiler_params=pltpu.CompilerParams(collective_id=0),
    )(x)


# device time: 11897 ns/iter; 1.0066x vs baseline; 1.0066x over previous
import jax
import jax.numpy as jnp
from jax import lax
from jax.experimental import pallas as pl
from jax.experimental.pallas import tpu as pltpu

N_DEV = 8
N_ROUNDS = 3


def kernel(x):
    m, n = x.shape

    def body(x_ref, out_ref, send_ref, recv_ref, send_sems, recv_sems):
        my = lax.axis_index("i")
        partners = [jnp.bitwise_xor(my, 1 << k) for k in range(N_ROUNDS)]

        barrier_sem = pltpu.get_barrier_semaphore()
        for p in partners:
            pl.semaphore_signal(
                barrier_sem, inc=1,
                device_id=(p,), device_id_type=pl.DeviceIdType.MESH,
            )

        def rdma(k):
            return pltpu.make_async_remote_copy(
                src_ref=send_ref.at[k],
                dst_ref=recv_ref.at[k],
                send_sem=send_sems.at[k],
                recv_sem=recv_sems.at[k],
                device_id=(partners[k],),
                device_id_type=pl.DeviceIdType.MESH,
            )

        t = x_ref[:, :].astype(jnp.float32)
        h = m
        while h > 1:
            h //= 2
            t = t[:h, :] * t[h:, :]
        send_ref[0, :, :] = t
        pl.semaphore_wait(barrier_sem, N_ROUNDS)

        r0 = rdma(0)
        r0.start()

        lc = x_ref[:, :].astype(jnp.float32)
        s = 1
        while s < 32:
            shifted = jnp.concatenate(
                [jnp.ones((s, n), jnp.float32), lc[:-s, :]], axis=0
            )
            lc = lc * shifted
            s *= 2

        r0.wait_recv()
        send_ref[1, :, :] = send_ref[0, :, :] * recv_ref[0, :, :]
        e0 = jnp.where(partners[0] < my, recv_ref[0, :, :], 1.0)
        r1 = rdma(1)
        r1.start()

        while s < m:
            shifted = jnp.concatenate(
                [jnp.ones((s, n), jnp.float32), lc[:-s, :]], axis=0
            )
            lc = lc * shifted
            s *= 2
        lc = lc * e0

        r1.wait_recv()
        send_ref[2, :, :] = send_ref[1, :, :] * recv_ref[1, :, :]
        e1 = jnp.where(partners[1] < my, recv_ref[1, :, :], 1.0)
        r2 = rdma(2)
        r2.start()

        lc = lc * e1

        r2.wait_recv()
        e2 = jnp.where(partners[2] < my, recv_ref[2, :, :], 1.0)
        out_ref[:, :] = lc * e2

        r0.wait_send()
        r1.wait_send()
        r2.wait_send()

    return pl.pallas_call(
        body,
        out_shape=jax.ShapeDtypeStruct((m, n), jnp.float32),
        in_specs=[pl.BlockSpec(memory_space=pltpu.VMEM)],
        out_specs=pl.BlockSpec(memory_space=pltpu.VMEM),
        scratch_shapes=[
            pltpu.VMEM((N_ROUNDS, 1, n), jnp.float32),
            pltpu.VMEM((N_ROUNDS, 1, n), jnp.float32),
            pltpu.SemaphoreType.DMA((N_ROUNDS,)),
            pltpu.SemaphoreType.DMA((N_ROUNDS,)),
        ],
        compiler_params=pltpu.CompilerParams(collective_id=0),
    )(x)
